# baseline (speedup 1.0000x reference)
import functools

import jax
import jax.numpy as jnp
from jax import lax
from jax.experimental import pallas as pl
from jax.experimental.pallas import tpu as pltpu
from jax.experimental.pallas import tpu_sc as plsc

N = 2048
B = 2
IN = 64
CH = 64
HALF = CH // 2
R = B * CH
NOUT = N // 2

HT_BLK = 256
CNT_BLK = 256

J_TC = 0
J_SC = NOUT - J_TC
MP_JBLK = 128
MP_KBLK = 256

NTILES = 32
JPT = J_SC // NTILES
NQ = 8
KQ = N // NQ
L = 16
NCH = KQ // L
NCHUNK = N // L
NG = R // L


def _ht_kernel(adj_ref, x_ref, xb_ref, wl_ref, bl_ref, we_ref, be_ref, out_ref):
    adj_blk = adj_ref[...]
    for b in range(B):
        z = jax.lax.dot_general(
            x_ref[b], wl_ref[...],
            (((1,), (1,)), ((), ())),
            preferred_element_type=jnp.float32)
        lin = jax.lax.dot_general(
            adj_blk, z,
            (((1,), (0,)), ((), ())),
            preferred_element_type=jnp.float32)
        lin = lin + bl_ref[...][None, :]
        eye = jax.lax.dot_general(
            xb_ref[b], we_ref[...],
            (((1,), (1,)), ((), ())),
            preferred_element_type=jnp.float32)
        eye = eye + be_ref[...][None, :]
        out_ref[:, b * CH:b * CH + HALF] = lin
        out_ref[:, b * CH + HALF:(b + 1) * CH] = eye


def _cnt_kernel(adj_ref, out_ref):
    nz = (adj_ref[...] != 0.0).astype(jnp.float32)
    kk = jax.lax.broadcasted_iota(jnp.int32, (N, NCHUNK), 0) // L
    cc = jax.lax.broadcasted_iota(jnp.int32, (N, NCHUNK), 1)
    sel = (kk == cc).astype(jnp.float32)
    cnt = jax.lax.dot_general(
        nz, sel, (((1,), (0,)), ((), ())),
        preferred_element_type=jnp.float32)
    out_ref[...] = cnt.astype(jnp.int32)


def _maxpool_tc_kernel(adj_ref, ht_ref, out_ref):
    @pl.when(pl.program_id(1) == 0)
    def _init():
        out_ref[...] = jnp.full((MP_JBLK, R), -jnp.inf, dtype=jnp.float32)

    a = adj_ref[...]
    h = ht_ref[...]
    acc = out_ref[...]
    for k in range(MP_KBLK):
        acc = jnp.maximum(acc, a[:, k:k + 1] * h[k:k + 1, :])
    out_ref[...] = acc


def _sc_maxpool_body(adj_hbm, ht_hbm, cnt_hbm, out_hbm,
                     ht0_v, ht1_v, ar0_v, ar1_v, cnt_v, acc_v, nz_v,
                     sem0, sem1):
    nc = 2
    wid = lax.axis_index("s") * nc + lax.axis_index("c")
    j0 = J_TC + wid * JPT
    lanes = lax.iota(jnp.int32, L)
    neg_inf = jnp.full((L,), -jnp.inf, dtype=jnp.float32)
    zeros_i = jnp.zeros((L,), dtype=jnp.int32)

    ht_bufs = [ht0_v, ht1_v]
    ar_bufs = [ar0_v, ar1_v]
    sems = [sem0, sem1]

    def start_slice(q):
        h1 = pltpu.async_copy(
            ht_hbm.at[pl.ds(q * KQ * R, KQ * R)], ht_bufs[q % 2], sems[q % 2])
        h2 = pltpu.async_copy(
            adj_hbm.at[pl.ds(j0, JPT), pl.ds(q * KQ, KQ)],
            ar_bufs[q % 2], sems[q % 2])
        return (h1, h2)

    pending = start_slice(0)
    pltpu.sync_copy(cnt_hbm.at[pl.ds(j0, JPT), :], cnt_v)

    for q in range(NQ):
        if q + 1 < NQ:
            nxt = start_slice(q + 1)
        pending[0].wait()
        pending[1].wait()
        ht_v = ht_bufs[q % 2]
        arow_v = ar_bufs[q % 2]

        def row_body(row, _, q=q, ht_v=ht_v, arow_v=arow_v):
            row_splat = jnp.broadcast_to(row, (L,)).astype(jnp.int32)
            if q == 0:
                accs = [neg_inf for _ in range(NG)]
                nzv = zeros_i
            else:
                accs = [acc_v[pl.ds(row * R + g * L, L)] for g in range(NG)]
                nzv = nz_v[pl.ds(row * L, L)]

            cvec = cnt_v[row, pl.ds(q * NCH, L)]
            nzv = nzv + cvec
            om = cvec > 0

            def c_cond(c):
                return jnp.any(c[0])

            def c_body(c):
                om = c[0]
                caccs = list(c[1:])
                ffs_c = plsc.all_reduce_ffs(om)
                col = ffs_c * L + lanes
                av = plsc.load_gather(arow_v, [row_splat, col])
                m = av != 0.0

                def e_cond(e):
                    return jnp.any(e[0])

                def e_body(e, chq=ffs_c):
                    em = e[0]
                    eaccs = list(e[1:])
                    ffs_e = plsc.all_reduce_ffs(em)
                    cole = chq * L + ffs_e
                    vsp = plsc.load_gather(arow_v, [row_splat, cole])
                    kbase = cole * R
                    new = []
                    for g in range(NG):
                        hv = plsc.load_gather(ht_v, [kbase + (g * L) + lanes])
                        new.append(jnp.maximum(eaccs[g], vsp * hv))
                    em = jnp.logical_and(em, lanes != ffs_e)
                    return (em, *new)

                res = lax.while_loop(e_cond, e_body, (m, *caccs))
                om = jnp.logical_and(om, lanes != ffs_c)
                return (om, *res[1:])

            out = lax.while_loop(c_cond, c_body, (om, *accs))
            accs = list(out[1:])

            if q == NQ - 1:
                nnz = jnp.sum(nzv)
                hz = nnz < N
                accs = [jnp.where(hz, jnp.maximum(a, 0.0), a) for a in accs]
            for g in range(NG):
                acc_v[pl.ds(row * R + g * L, L)] = accs[g]
            if q != NQ - 1:
                nz_v[pl.ds(row * L, L)] = nzv
            return 0

        lax.fori_loop(0, JPT, row_body, 0)
        if q + 1 < NQ:
            pending = nxt

    pltpu.sync_copy(acc_v, out_hbm.at[pl.ds(wid * JPT * R, JPT * R)])


_sc_maxpool = functools.partial(
    pl.kernel,
    out_type=jax.ShapeDtypeStruct((J_SC * R,), jnp.float32),
    mesh=plsc.VectorSubcoreMesh(core_axis_name="c", subcore_axis_name="s"),
    scratch_types=[
        pltpu.VMEM((KQ * R,), jnp.float32),
        pltpu.VMEM((KQ * R,), jnp.float32),
        pltpu.VMEM((JPT, KQ), jnp.float32),
        pltpu.VMEM((JPT, KQ), jnp.float32),
        pltpu.VMEM((JPT, NCHUNK), jnp.int32),
        pltpu.VMEM((JPT * R,), jnp.float32),
        pltpu.VMEM((JPT * L,), jnp.int32),
        pltpu.SemaphoreType.DMA,
        pltpu.SemaphoreType.DMA,
    ],
    compiler_params=pltpu.CompilerParams(needs_layout_passes=False),
)(_sc_maxpool_body)


@jax.jit
def kernel(x, adj, W_lin, b_lin, W_eye, b_eye):
    hT = pl.pallas_call(
        _ht_kernel,
        grid=(N // HT_BLK,),
        in_specs=[
            pl.BlockSpec((HT_BLK, N), lambda i: (i, 0)),
            pl.BlockSpec((B, N, IN), lambda i: (0, 0, 0)),
            pl.BlockSpec((B, HT_BLK, IN), lambda i: (0, i, 0)),
            pl.BlockSpec((HALF, IN), lambda i: (0, 0)),
            pl.BlockSpec((HALF,), lambda i: (0,)),
            pl.BlockSpec((HALF, IN), lambda i: (0, 0)),
            pl.BlockSpec((HALF,), lambda i: (0,)),
        ],
        out_specs=pl.BlockSpec((HT_BLK, R), lambda i: (i, 0)),
        out_shape=jax.ShapeDtypeStruct((N, R), jnp.float32),
    )(adj, x, x, W_lin, b_lin, W_eye, b_eye)

    counts = pl.pallas_call(
        _cnt_kernel,
        grid=(NOUT // CNT_BLK,),
        in_specs=[pl.BlockSpec((CNT_BLK, N), lambda i: (i, 0))],
        out_specs=pl.BlockSpec((CNT_BLK, NCHUNK), lambda i: (i, 0)),
        out_shape=jax.ShapeDtypeStruct((NOUT, NCHUNK), jnp.int32),
    )(adj)

    out_sc = _sc_maxpool(adj, hT.reshape(-1), counts).reshape(J_SC, R)

    if J_TC:
        out_tc = pl.pallas_call(
            _maxpool_tc_kernel,
            grid=(J_TC // MP_JBLK, N // MP_KBLK),
            in_specs=[
                pl.BlockSpec((MP_JBLK, MP_KBLK), lambda i, k: (i, k)),
                pl.BlockSpec((MP_KBLK, R), lambda i, k: (k, 0)),
            ],
            out_specs=pl.BlockSpec((MP_JBLK, R), lambda i, k: (i, 0)),
            out_shape=jax.ShapeDtypeStruct((J_TC, R), jnp.float32),
        )(adj[:J_TC], hT)
        outT = jnp.concatenate([out_tc, out_sc], axis=0)
    else:
        outT = out_sc
    return jnp.transpose(outT.reshape(NOUT, B, CH), (1, 0, 2))


# speedup vs baseline: 1.0032x; 1.0032x over previous
import functools

import jax
import jax.numpy as jnp
from jax import lax
from jax.experimental import pallas as pl
from jax.experimental.pallas import tpu as pltpu
from jax.experimental.pallas import tpu_sc as plsc

N = 2048
B = 2
IN = 64
CH = 64
HALF = CH // 2
R = B * CH
NOUT = N // 2

HT_BLK = 256
CNT_BLK = 256

J_TC = 0
J_SC = NOUT - J_TC
MP_JBLK = 128
MP_KBLK = 256

NTILES = 32
JPT = J_SC // NTILES
NQ = 8
KQ = N // NQ
L = 16
NCH = KQ // L
NCHUNK = N // L
NG = R // L


def _ht_kernel(adj_ref, x_ref, xb_ref, wl_ref, bl_ref, we_ref, be_ref,
               out_ref, cnt_ref):
    adj_blk = adj_ref[...]
    nz = (adj_blk != 0.0).astype(jnp.float32)
    kk = jax.lax.broadcasted_iota(jnp.int32, (N, NCHUNK), 0) // L
    cc = jax.lax.broadcasted_iota(jnp.int32, (N, NCHUNK), 1)
    sel = (kk == cc).astype(jnp.float32)
    cnt = jax.lax.dot_general(
        nz, sel, (((1,), (0,)), ((), ())),
        preferred_element_type=jnp.float32)
    cnt_ref[...] = cnt.astype(jnp.int32)
    for b in range(B):
        z = jax.lax.dot_general(
            x_ref[b], wl_ref[...],
            (((1,), (1,)), ((), ())),
            preferred_element_type=jnp.float32)
        lin = jax.lax.dot_general(
            adj_blk, z,
            (((1,), (0,)), ((), ())),
            preferred_element_type=jnp.float32)
        lin = lin + bl_ref[...][None, :]
        eye = jax.lax.dot_general(
            xb_ref[b], we_ref[...],
            (((1,), (1,)), ((), ())),
            preferred_element_type=jnp.float32)
        eye = eye + be_ref[...][None, :]
        out_ref[:, b * CH:b * CH + HALF] = lin
        out_ref[:, b * CH + HALF:(b + 1) * CH] = eye


def _maxpool_tc_kernel(adj_ref, ht_ref, out_ref):
    @pl.when(pl.program_id(1) == 0)
    def _init():
        out_ref[...] = jnp.full((MP_JBLK, R), -jnp.inf, dtype=jnp.float32)

    a = adj_ref[...]
    h = ht_ref[...]
    acc = out_ref[...]
    for k in range(MP_KBLK):
        acc = jnp.maximum(acc, a[:, k:k + 1] * h[k:k + 1, :])
    out_ref[...] = acc


def _sc_maxpool_body(adj_hbm, ht_hbm, cnt_hbm, out_hbm,
                     ht0_v, ht1_v, ar0_v, ar1_v, cnt_v, acc_v, nz_v,
                     sem0, sem1):
    nc = 2
    wid = lax.axis_index("s") * nc + lax.axis_index("c")
    j0 = J_TC + wid * JPT
    lanes = lax.iota(jnp.int32, L)
    neg_inf = jnp.full((L,), -jnp.inf, dtype=jnp.float32)
    zeros_i = jnp.zeros((L,), dtype=jnp.int32)

    ht_bufs = [ht0_v, ht1_v]
    ar_bufs = [ar0_v, ar1_v]
    sems = [sem0, sem1]

    def start_slice(q):
        h1 = pltpu.async_copy(
            ht_hbm.at[pl.ds(q * KQ * R, KQ * R)], ht_bufs[q % 2], sems[q % 2])
        h2 = pltpu.async_copy(
            adj_hbm.at[pl.ds(j0, JPT), pl.ds(q * KQ, KQ)],
            ar_bufs[q % 2], sems[q % 2])
        return (h1, h2)

    pending = start_slice(0)
    pltpu.sync_copy(cnt_hbm.at[pl.ds(j0, JPT), :], cnt_v)

    for q in range(NQ):
        if q + 1 < NQ:
            nxt = start_slice(q + 1)
        pending[0].wait()
        pending[1].wait()
        ht_v = ht_bufs[q % 2]
        arow_v = ar_bufs[q % 2]

        def row_body(row, _, q=q, ht_v=ht_v, arow_v=arow_v):
            row_splat = jnp.broadcast_to(row, (L,)).astype(jnp.int32)
            if q == 0:
                accs = [neg_inf for _ in range(NG)]
                nzv = zeros_i
            else:
                accs = [acc_v[pl.ds(row * R + g * L, L)] for g in range(NG)]
                nzv = nz_v[pl.ds(row * L, L)]

            cvec = cnt_v[row, pl.ds(q * NCH, L)]
            nzv = nzv + cvec
            om = cvec > 0

            def c_cond(c):
                return jnp.any(c[0])

            def c_body(c):
                om = c[0]
                caccs = list(c[1:])
                ffs_c = plsc.all_reduce_ffs(om)
                col = ffs_c * L + lanes
                av = plsc.load_gather(arow_v, [row_splat, col])
                m = av != 0.0

                def e_cond(e):
                    return jnp.any(e[0])

                def e_body(e, chq=ffs_c):
                    em = e[0]
                    eaccs = list(e[1:])
                    ffs_e = plsc.all_reduce_ffs(em)
                    cole = chq * L + ffs_e
                    vsp = plsc.load_gather(arow_v, [row_splat, cole])
                    kbase = cole * R
                    new = []
                    for g in range(NG):
                        hv = plsc.load_gather(ht_v, [kbase + (g * L) + lanes])
                        new.append(jnp.maximum(eaccs[g], vsp * hv))
                    em = jnp.logical_and(em, lanes != ffs_e)
                    return (em, *new)

                res = lax.while_loop(e_cond, e_body, (m, *caccs))
                om = jnp.logical_and(om, lanes != ffs_c)
                return (om, *res[1:])

            out = lax.while_loop(c_cond, c_body, (om, *accs))
            accs = list(out[1:])

            if q == NQ - 1:
                nnz = jnp.sum(nzv)
                hz = nnz < N
                accs = [jnp.where(hz, jnp.maximum(a, 0.0), a) for a in accs]
            for g in range(NG):
                acc_v[pl.ds(row * R + g * L, L)] = accs[g]
            if q != NQ - 1:
                nz_v[pl.ds(row * L, L)] = nzv
            return 0

        lax.fori_loop(0, JPT, row_body, 0)
        if q + 1 < NQ:
            pending = nxt

    pltpu.sync_copy(acc_v, out_hbm.at[pl.ds(wid * JPT * R, JPT * R)])


_sc_maxpool = functools.partial(
    pl.kernel,
    out_type=jax.ShapeDtypeStruct((J_SC * R,), jnp.float32),
    mesh=plsc.VectorSubcoreMesh(core_axis_name="c", subcore_axis_name="s"),
    scratch_types=[
        pltpu.VMEM((KQ * R,), jnp.float32),
        pltpu.VMEM((KQ * R,), jnp.float32),
        pltpu.VMEM((JPT, KQ), jnp.float32),
        pltpu.VMEM((JPT, KQ), jnp.float32),
        pltpu.VMEM((JPT, NCHUNK), jnp.int32),
        pltpu.VMEM((JPT * R,), jnp.float32),
        pltpu.VMEM((JPT * L,), jnp.int32),
        pltpu.SemaphoreType.DMA,
        pltpu.SemaphoreType.DMA,
    ],
    compiler_params=pltpu.CompilerParams(needs_layout_passes=False),
)(_sc_maxpool_body)


@jax.jit
def kernel(x, adj, W_lin, b_lin, W_eye, b_eye):
    hT, counts = pl.pallas_call(
        _ht_kernel,
        grid=(N // HT_BLK,),
        in_specs=[
            pl.BlockSpec((HT_BLK, N), lambda i: (i, 0)),
            pl.BlockSpec((B, N, IN), lambda i: (0, 0, 0)),
            pl.BlockSpec((B, HT_BLK, IN), lambda i: (0, i, 0)),
            pl.BlockSpec((HALF, IN), lambda i: (0, 0)),
            pl.BlockSpec((HALF,), lambda i: (0,)),
            pl.BlockSpec((HALF, IN), lambda i: (0, 0)),
            pl.BlockSpec((HALF,), lambda i: (0,)),
        ],
        out_specs=[
            pl.BlockSpec((HT_BLK, R), lambda i: (i, 0)),
            pl.BlockSpec((HT_BLK, NCHUNK), lambda i: (i, 0)),
        ],
        out_shape=[
            jax.ShapeDtypeStruct((N, R), jnp.float32),
            jax.ShapeDtypeStruct((N, NCHUNK), jnp.int32),
        ],
    )(adj, x, x, W_lin, b_lin, W_eye, b_eye)

    out_sc = _sc_maxpool(adj, hT.reshape(-1), counts).reshape(J_SC, R)

    if J_TC:
        out_tc = pl.pallas_call(
            _maxpool_tc_kernel,
            grid=(J_TC // MP_JBLK, N // MP_KBLK),
            in_specs=[
                pl.BlockSpec((MP_JBLK, MP_KBLK), lambda i, k: (i, k)),
                pl.BlockSpec((MP_KBLK, R), lambda i, k: (k, 0)),
            ],
            out_specs=pl.BlockSpec((MP_JBLK, R), lambda i, k: (i, 0)),
            out_shape=jax.ShapeDtypeStruct((J_TC, R), jnp.float32),
        )(adj[:J_TC], hT)
        outT = jnp.concatenate([out_tc, out_sc], axis=0)
    else:
        outT = out_sc
    return jnp.transpose(outT.reshape(NOUT, B, CH), (1, 0, 2))


# speedup vs baseline: 1.0199x; 1.0167x over previous
import functools

import jax
import jax.numpy as jnp
from jax import lax
from jax.experimental import pallas as pl
from jax.experimental.pallas import tpu as pltpu
from jax.experimental.pallas import tpu_sc as plsc

N = 2048
B = 2
IN = 64
CH = 64
HALF = CH // 2
R = B * CH
NOUT = N // 2

HT_BLK = 256

NTILES = 32
JPT = NOUT // NTILES
NQ = 8
KQ = N // NQ
L = 16
NCH = KQ // L
NCHUNK = N // L
NG = R // L


def _ht_kernel(adj_ref, x_ref, xb_ref, wl_ref, bl_ref, we_ref, be_ref,
               out_ref, cnt_ref):
    adj_blk = adj_ref[...]

    @pl.when(pl.program_id(0) < NOUT // HT_BLK)
    def _counts():
        nz = (adj_blk != 0.0).astype(jnp.bfloat16)
        kk = jax.lax.broadcasted_iota(jnp.int32, (N, NCHUNK), 0) // L
        cc = jax.lax.broadcasted_iota(jnp.int32, (N, NCHUNK), 1)
        sel = (kk == cc).astype(jnp.bfloat16)
        cnt = jax.lax.dot_general(
            nz, sel, (((1,), (0,)), ((), ())),
            preferred_element_type=jnp.float32)
        cnt_ref[...] = cnt.astype(jnp.int32)
    for b in range(B):
        z = jax.lax.dot_general(
            x_ref[b], wl_ref[...],
            (((1,), (1,)), ((), ())),
            preferred_element_type=jnp.float32)
        lin = jax.lax.dot_general(
            adj_blk, z,
            (((1,), (0,)), ((), ())),
            preferred_element_type=jnp.float32)
        lin = lin + bl_ref[...][None, :]
        eye = jax.lax.dot_general(
            xb_ref[b], we_ref[...],
            (((1,), (1,)), ((), ())),
            preferred_element_type=jnp.float32)
        eye = eye + be_ref[...][None, :]
        out_ref[:, b * CH:b * CH + HALF] = lin
        out_ref[:, b * CH + HALF:(b + 1) * CH] = eye


def _sc_maxpool_body(adj_hbm, ht_hbm, cnt_hbm, out_hbm,
                     ht0_v, ht1_v, ar0_v, ar1_v, cnt_v, acc0_v, acc1_v, nz_v,
                     sem0, sem1):
    nc = 2
    wid = lax.axis_index("s") * nc + lax.axis_index("c")
    j0 = wid * JPT
    acc_bufs = [acc0_v, acc1_v]
    lanes = lax.iota(jnp.int32, L)
    neg_inf = jnp.full((L,), -jnp.inf, dtype=jnp.float32)
    zeros_i = jnp.zeros((L,), dtype=jnp.int32)

    ht_bufs = [ht0_v, ht1_v]
    ar_bufs = [ar0_v, ar1_v]
    sems = [sem0, sem1]

    def start_slice(q):
        h1 = pltpu.async_copy(
            ht_hbm.at[pl.ds(q * KQ * R, KQ * R)], ht_bufs[q % 2], sems[q % 2])
        h2 = pltpu.async_copy(
            adj_hbm.at[pl.ds(j0, JPT), pl.ds(q * KQ, KQ)],
            ar_bufs[q % 2], sems[q % 2])
        return (h1, h2)

    pending = start_slice(0)
    pltpu.sync_copy(cnt_hbm.at[pl.ds(j0, JPT), :], cnt_v)

    for q in range(NQ):
        if q + 1 < NQ:
            nxt = start_slice(q + 1)
        pending[0].wait()
        pending[1].wait()
        ht_v = ht_bufs[q % 2]
        arow_v = ar_bufs[q % 2]

        def row_body(row, _, q=q, ht_v=ht_v, arow_v=arow_v):
            row_splat = jnp.broadcast_to(row, (L,)).astype(jnp.int32)
            if q == 0:
                accs = [neg_inf for _ in range(NG)]
                nzv = zeros_i
            else:
                accs = [acc_bufs[g // 4][row, pl.ds((g % 4) * L, L)]
                        for g in range(NG)]
                nzv = nz_v[pl.ds(row * L, L)]

            cvec = cnt_v[row, pl.ds(q * NCH, L)]
            nzv = nzv + cvec
            om = cvec > 0

            def c_cond(c):
                return jnp.any(c[0])

            def c_body(c):
                om = c[0]
                caccs = list(c[1:])
                ffs_c = plsc.all_reduce_ffs(om)
                col = ffs_c * L + lanes
                av = plsc.load_gather(arow_v, [row_splat, col])
                m = av != 0.0

                def e_cond(e):
                    return jnp.any(e[0])

                def e_body(e, chq=ffs_c):
                    em = e[0]
                    eaccs = list(e[1:])
                    ffs_e = plsc.all_reduce_ffs(em)
                    cole = chq * L + ffs_e
                    vsp = plsc.load_gather(arow_v, [row_splat, cole])
                    kbase = cole * R
                    new = []
                    for g in range(NG):
                        hv = plsc.load_gather(ht_v, [kbase + (g * L) + lanes])
                        new.append(jnp.maximum(eaccs[g], vsp * hv))
                    em = jnp.logical_and(em, lanes != ffs_e)
                    return (em, *new)

                res = lax.while_loop(e_cond, e_body, (m, *caccs))
                om = jnp.logical_and(om, lanes != ffs_c)
                return (om, *res[1:])

            out = lax.while_loop(c_cond, c_body, (om, *accs))
            accs = list(out[1:])

            if q == NQ - 1:
                nnz = jnp.sum(nzv)
                hz = nnz < N
                accs = [jnp.where(hz, jnp.maximum(a, 0.0), a) for a in accs]
            for g in range(NG):
                acc_bufs[g // 4][row, pl.ds((g % 4) * L, L)] = accs[g]
            if q != NQ - 1:
                nz_v[pl.ds(row * L, L)] = nzv
            return 0

        lax.fori_loop(0, JPT, row_body, 0)
        if q + 1 < NQ:
            pending = nxt

    pltpu.sync_copy(acc0_v, out_hbm.at[0, pl.ds(j0, JPT), :])
    pltpu.sync_copy(acc1_v, out_hbm.at[1, pl.ds(j0, JPT), :])


_sc_maxpool = functools.partial(
    pl.kernel,
    out_type=jax.ShapeDtypeStruct((B, NOUT, CH), jnp.float32),
    mesh=plsc.VectorSubcoreMesh(core_axis_name="c", subcore_axis_name="s"),
    scratch_types=[
        pltpu.VMEM((KQ * R,), jnp.float32),
        pltpu.VMEM((KQ * R,), jnp.float32),
        pltpu.VMEM((JPT, KQ), jnp.float32),
        pltpu.VMEM((JPT, KQ), jnp.float32),
        pltpu.VMEM((JPT, NCHUNK), jnp.int32),
        pltpu.VMEM((JPT, CH), jnp.float32),
        pltpu.VMEM((JPT, CH), jnp.float32),
        pltpu.VMEM((JPT * L,), jnp.int32),
        pltpu.SemaphoreType.DMA,
        pltpu.SemaphoreType.DMA,
    ],
    compiler_params=pltpu.CompilerParams(needs_layout_passes=False),
)(_sc_maxpool_body)


@jax.jit
def kernel(x, adj, W_lin, b_lin, W_eye, b_eye):
    hT, counts = pl.pallas_call(
        _ht_kernel,
        grid=(N // HT_BLK,),
        in_specs=[
            pl.BlockSpec((HT_BLK, N), lambda i: (i, 0)),
            pl.BlockSpec((B, N, IN), lambda i: (0, 0, 0)),
            pl.BlockSpec((B, HT_BLK, IN), lambda i: (0, i, 0)),
            pl.BlockSpec((HALF, IN), lambda i: (0, 0)),
            pl.BlockSpec((HALF,), lambda i: (0,)),
            pl.BlockSpec((HALF, IN), lambda i: (0, 0)),
            pl.BlockSpec((HALF,), lambda i: (0,)),
        ],
        out_specs=[
            pl.BlockSpec((HT_BLK, R), lambda i: (i, 0)),
            pl.BlockSpec((HT_BLK, NCHUNK), lambda i: (i, 0)),
        ],
        out_shape=[
            jax.ShapeDtypeStruct((N, R), jnp.float32),
            jax.ShapeDtypeStruct((N, NCHUNK), jnp.int32),
        ],
    )(adj, x, x, W_lin, b_lin, W_eye, b_eye)

    return _sc_maxpool(adj, hT.reshape(-1), counts)
